# baseline (speedup 1.0000x reference)
import jax
import jax.numpy as jnp
from jax.experimental import pallas as pl
from jax.experimental.pallas import tpu as pltpu

DIM = 768
NUM_EXPERTS = 8
NUM_MICROS = 16
TOP_K = 2
TOP_K_MICROS = 8
EXPERT_DIM = 1536
MICRO_HID = DIM // 2
SEQ = 2048
TILE = 512
NUM_TILES = SEQ // TILE
EG = 2
MG = 4
E_STEPS = NUM_EXPERTS // EG
M_STEPS = NUM_MICROS // MG
EPS = 1e-5


def _gelu(v):
    return 0.5 * v * (1.0 + jax.lax.erf(v * 0.7071067811865476))


def _layer_norm(v, g, b):
    mu = jnp.mean(v, axis=-1, keepdims=True)
    var = jnp.mean((v - mu) ** 2, axis=-1, keepdims=True)
    return (v - mu) * jax.lax.rsqrt(var + EPS) * g + b


def _topk_mask_combine(probs, k):
    work = probs
    thr = None
    sel_sum = jnp.zeros(probs.shape[:-1] + (1,), probs.dtype)
    for _ in range(k):
        thr = jnp.max(work, axis=-1, keepdims=True)
        sel_sum = sel_sum + thr
        work = jnp.where(work >= thr, -jnp.inf, work)
    mask = probs >= thr
    return jnp.where(mask, probs, 0.0) / (sel_sum + 1e-8)


def _col(combine, idx):
    lane = jax.lax.broadcasted_iota(jnp.int32, combine.shape, 1)
    return jnp.sum(jnp.where(lane == idx, combine, 0.0), axis=-1,
                   keepdims=True)


def _expert_kernel(x_ref, xb_ref, rw_ref, rb_ref, w1_ref, b1_ref, w2_ref,
                   b2_ref, out_ref, cmb_ref):
    g = pl.program_id(0)
    t = pl.program_id(1)

    @pl.when(g == 0)
    def _router():
        xt = x_ref[pl.ds(t * TILE, TILE), :]
        logits = jnp.dot(xt, rw_ref[...], preferred_element_type=jnp.float32)
        logits = logits + rb_ref[...]
        probs = jax.nn.softmax(logits, axis=-1)
        cmb_ref[pl.ds(t * TILE, TILE), :] = _topk_mask_combine(probs, TOP_K)

    combine = cmb_ref[pl.ds(t * TILE, TILE), :]
    xbt = xb_ref[pl.ds(t * TILE, TILE), :]

    acc = None
    for j in range(EG):
        h = jnp.dot(xbt, w1_ref[0, j], preferred_element_type=jnp.float32)
        h = _gelu(h + b1_ref[0, j])
        eo = jnp.dot(h.astype(jnp.bfloat16), w2_ref[0, j],
                     preferred_element_type=jnp.float32)
        eo = (eo + b2_ref[0, j]) * _col(combine, g * EG + j)
        acc = eo if acc is None else acc + eo

    @pl.when(g == 0)
    def _init():
        out_ref[pl.ds(t * TILE, TILE), :] = acc

    @pl.when(g > 0)
    def _acc():
        out_ref[pl.ds(t * TILE, TILE), :] += acc


def _micro_kernel(eo_ref, rw_ref, rb_ref, w1_ref, b1_ref, w2_ref, b2_ref,
                  lng_ref, lnb_ref, ng_ref, nb_ref, out_ref, acc_ref,
                  cmb_ref):
    g = pl.program_id(0)
    t = pl.program_id(1)
    xt = eo_ref[pl.ds(t * TILE, TILE), :]

    @pl.when(g == 0)
    def _router():
        logits = jnp.dot(xt, rw_ref[...], preferred_element_type=jnp.float32)
        logits = logits + rb_ref[...]
        probs = jax.nn.softmax(logits, axis=-1)
        cmb_ref[pl.ds(t * TILE, TILE), :] = _topk_mask_combine(
            probs, TOP_K_MICROS)

    mcombine = cmb_ref[pl.ds(t * TILE, TILE), :]
    xbt = xt.astype(jnp.bfloat16)

    acc = None
    for j in range(MG):
        mh = jnp.dot(xbt, w1_ref[0, j], preferred_element_type=jnp.float32)
        mh = _gelu(mh + b1_ref[0, j])
        mf = jnp.dot(mh.astype(jnp.bfloat16), w2_ref[0, j],
                     preferred_element_type=jnp.float32)
        pre = xt + mf + b2_ref[0, j]
        mo = _layer_norm(pre, lng_ref[0, j], lnb_ref[0, j])
        mo = mo * _col(mcombine, g * MG + j)
        acc = mo if acc is None else acc + mo

    @pl.when(g == 0)
    def _init():
        acc_ref[pl.ds(t * TILE, TILE), :] = acc

    @pl.when(g > 0)
    def _acc():
        acc_ref[pl.ds(t * TILE, TILE), :] += acc

    @pl.when(g == M_STEPS - 1)
    def _final():
        combined = xt + 0.1 * acc_ref[pl.ds(t * TILE, TILE), :]
        out_ref[pl.ds(t * TILE, TILE), :] = _layer_norm(
            combined, ng_ref[...], nb_ref[...])


def _resident(shape):
    return pl.BlockSpec(shape, lambda *_: tuple(0 for _ in shape))


def _per_g(shape):
    return pl.BlockSpec(shape, lambda g, t: (g,) + tuple(0 for _ in shape[1:]))


@jax.jit
def kernel(x, router_W, router_b, expert_W1, expert_b1, expert_W2, expert_b2,
           micro_router_W, micro_router_b, micro_W1, micro_b1, micro_W2,
           micro_b2, micro_ln_g, micro_ln_b, norm_g, norm_b):
    B, S, D = x.shape
    xf = x.reshape(S, D)

    expert_output = pl.pallas_call(
        _expert_kernel,
        grid=(E_STEPS, NUM_TILES),
        in_specs=[
            _resident((S, D)),
            _resident((S, D)),
            _resident((D, NUM_EXPERTS)),
            _resident((1, NUM_EXPERTS)),
            _per_g((1, EG, DIM, EXPERT_DIM)),
            _per_g((1, EG, 1, EXPERT_DIM)),
            _per_g((1, EG, EXPERT_DIM, DIM)),
            _per_g((1, EG, DIM)),
        ],
        out_specs=_resident((S, D)),
        out_shape=jax.ShapeDtypeStruct((S, D), jnp.float32),
        scratch_shapes=[pltpu.VMEM((S, NUM_EXPERTS), jnp.float32)],
        compiler_params=pltpu.CompilerParams(
            dimension_semantics=("arbitrary", "arbitrary"),
        ),
    )(xf, xf.astype(jnp.bfloat16), router_W, router_b.reshape(1, -1),
      expert_W1.reshape(E_STEPS, EG, DIM, EXPERT_DIM).astype(jnp.bfloat16),
      expert_b1.reshape(E_STEPS, EG, 1, EXPERT_DIM),
      expert_W2.reshape(E_STEPS, EG, EXPERT_DIM, DIM).astype(jnp.bfloat16),
      expert_b2.reshape(E_STEPS, EG, DIM))

    out = pl.pallas_call(
        _micro_kernel,
        grid=(M_STEPS, NUM_TILES),
        in_specs=[
            _resident((S, D)),
            _resident((D, NUM_MICROS)),
            _resident((1, NUM_MICROS)),
            _per_g((1, MG, DIM, MICRO_HID)),
            _per_g((1, MG, 1, MICRO_HID)),
            _per_g((1, MG, MICRO_HID, DIM)),
            _per_g((1, MG, DIM)),
            _per_g((1, MG, DIM)),
            _per_g((1, MG, DIM)),
            _resident((1, DIM)),
            _resident((1, DIM)),
        ],
        out_specs=_resident((S, D)),
        out_shape=jax.ShapeDtypeStruct((S, D), jnp.float32),
        scratch_shapes=[pltpu.VMEM((S, D), jnp.float32),
                        pltpu.VMEM((S, NUM_MICROS), jnp.float32)],
        compiler_params=pltpu.CompilerParams(
            dimension_semantics=("arbitrary", "arbitrary"),
        ),
    )(expert_output, micro_router_W, micro_router_b.reshape(1, -1),
      micro_W1.reshape(M_STEPS, MG, DIM, MICRO_HID).astype(jnp.bfloat16),
      micro_b1.reshape(M_STEPS, MG, 1, MICRO_HID),
      micro_W2.reshape(M_STEPS, MG, MICRO_HID, DIM).astype(jnp.bfloat16),
      micro_b2.reshape(M_STEPS, MG, DIM),
      micro_ln_g.reshape(M_STEPS, MG, DIM),
      micro_ln_b.reshape(M_STEPS, MG, DIM),
      norm_g.reshape(1, -1), norm_b.reshape(1, -1))

    return out.reshape(B, S, D)


# speedup vs baseline: 1.1103x; 1.1103x over previous
import jax
import jax.numpy as jnp
from jax import lax
from jax.experimental import pallas as pl
from jax.experimental.pallas import tpu as pltpu
from jax.experimental.pallas import tpu_sc as plsc

DIM = 768
NUM_EXPERTS = 8
NUM_MICROS = 16
TOP_K = 2
TOP_K_MICROS = 8
EXPERT_DIM = 1536
MICRO_HID = DIM // 2
SEQ = 2048
TILE = 512
NUM_TILES = SEQ // TILE
EG = 2
MG = 4
E_STEPS = NUM_EXPERTS // EG
M_STEPS = NUM_MICROS // MG
EPS = 1e-5


def _gelu(v):
    return 0.5 * v * (1.0 + jax.lax.erf(v * 0.7071067811865476))


def _layer_norm(v, g, b):
    mu = jnp.mean(v, axis=-1, keepdims=True)
    var = jnp.mean((v - mu) ** 2, axis=-1, keepdims=True)
    return (v - mu) * jax.lax.rsqrt(var + EPS) * g + b


def _topk_mask_combine(probs, k):
    work = probs
    thr = None
    sel_sum = jnp.zeros(probs.shape[:-1] + (1,), probs.dtype)
    for _ in range(k):
        thr = jnp.max(work, axis=-1, keepdims=True)
        sel_sum = sel_sum + thr
        work = jnp.where(work >= thr, -jnp.inf, work)
    mask = probs >= thr
    return jnp.where(mask, probs, 0.0) / (sel_sum + 1e-8)


def _col(combine, idx):
    lane = jax.lax.broadcasted_iota(jnp.int32, combine.shape, 1)
    return jnp.sum(jnp.where(lane == idx, combine, 0.0), axis=-1,
                   keepdims=True)


def _expert_kernel(x_ref, rw_ref, rb_ref, w1_ref, b1_ref, w2_ref,
                   b2_ref, out_ref, cmb_ref):
    g = pl.program_id(0)
    t = pl.program_id(1)
    xt = x_ref[pl.ds(t * TILE, TILE), :]

    @pl.when(g == 0)
    def _router():
        logits = jnp.dot(xt, rw_ref[...], preferred_element_type=jnp.float32)
        logits = logits + rb_ref[...]
        probs = jax.nn.softmax(logits, axis=-1)
        cmb_ref[pl.ds(t * TILE, TILE), :] = _topk_mask_combine(probs, TOP_K)

    combine = cmb_ref[pl.ds(t * TILE, TILE), :]

    acc = None
    for j in range(EG):
        h = jnp.dot(xt, w1_ref[0, j], preferred_element_type=jnp.float32)
        h = _gelu(h + b1_ref[0, j])
        eo = jnp.dot(h, w2_ref[0, j], preferred_element_type=jnp.float32)
        eo = (eo + b2_ref[0, j]) * _col(combine, g * EG + j)
        acc = eo if acc is None else acc + eo

    @pl.when(g == 0)
    def _init():
        out_ref[pl.ds(t * TILE, TILE), :] = acc

    @pl.when(g > 0)
    def _acc():
        out_ref[pl.ds(t * TILE, TILE), :] += acc


def _micro_kernel(z_ref, w0_ref, w1s_ref, rw_ref, rb_ref, w1_ref, b1_ref,
                  w2_ref, b2_ref, lng_ref, lnb_ref, ng_ref, nb_ref, out_ref,
                  eo_s, cmb_ref):
    g = pl.program_id(0)
    t = pl.program_id(1)

    @pl.when(g == 0)
    def _combine_experts():
        zt0 = z_ref[pl.ds(t * TILE, TILE), :]
        zt1 = z_ref[pl.ds(SEQ + t * TILE, TILE), :]
        w0 = w0_ref[pl.ds(t * TILE, TILE), :]
        w1 = w1s_ref[pl.ds(t * TILE, TILE), :]
        eo_s[pl.ds(t * TILE, TILE), :] = zt0 * w0 + zt1 * w1

    xt = eo_s[pl.ds(t * TILE, TILE), :]

    @pl.when(g == 0)
    def _router():
        logits = jnp.dot(xt, rw_ref[...], preferred_element_type=jnp.float32)
        logits = logits + rb_ref[...]
        probs = jax.nn.softmax(logits, axis=-1)
        cmb_ref[pl.ds(t * TILE, TILE), :] = _topk_mask_combine(
            probs, TOP_K_MICROS)

    mcombine = cmb_ref[pl.ds(t * TILE, TILE), :]

    acc = None
    for j in range(MG):
        mh = jnp.dot(xt, w1_ref[0, j], preferred_element_type=jnp.float32)
        mh = _gelu(mh + b1_ref[0, j])
        mf = jnp.dot(mh, w2_ref[0, j], preferred_element_type=jnp.float32)
        pre = xt + mf + b2_ref[0, j]
        mo = _layer_norm(pre, lng_ref[0, j], lnb_ref[0, j])
        mo = mo * _col(mcombine, g * MG + j)
        acc = mo if acc is None else acc + mo

    @pl.when(g == 0)
    def _init():
        out_ref[pl.ds(t * TILE, TILE), :] = acc

    @pl.when(g > 0)
    def _acc():
        out_ref[pl.ds(t * TILE, TILE), :] += acc

    @pl.when(g == M_STEPS - 1)
    def _final():
        combined = xt + 0.1 * out_ref[pl.ds(t * TILE, TILE), :]
        out_ref[pl.ds(t * TILE, TILE), :] = _layer_norm(
            combined, ng_ref[...], nb_ref[...])


TG = 256
NP = 2 * SEQ + NUM_EXPERTS * TG
NT = NP // TG
NW = 16
CHUNK = 2 * SEQ // NW
NV = CHUNK // 16


def _plan_kernel(x_ref, rw_ref, rb_ref, pos_ref, w0_ref, w1_ref, te_ref,
                 oh1_s, oh2_s, run_s, off_s):
    v = pl.program_id(0)
    t = v % NUM_TILES

    @pl.when(v == 0)
    def _init():
        run_s[...] = jnp.zeros((1, NUM_EXPERTS), jnp.float32)

    @pl.when(v < NUM_TILES)
    def _router():
        xt = x_ref[pl.ds(t * TILE, TILE), :]
        logits = jnp.dot(xt, rw_ref[...], preferred_element_type=jnp.float32)
        logits = logits + rb_ref[...]
        probs = jax.nn.softmax(logits, axis=-1)
        lane = jax.lax.broadcasted_iota(jnp.int32, probs.shape, 1)
        p1 = jnp.max(probs, axis=-1, keepdims=True)
        id1 = jnp.min(jnp.where(probs >= p1, lane, NUM_EXPERTS), axis=-1,
                      keepdims=True)
        masked = jnp.where(lane == id1, -jnp.inf, probs)
        p2 = jnp.max(masked, axis=-1, keepdims=True)
        id2 = jnp.min(jnp.where(masked >= p2, lane, NUM_EXPERTS), axis=-1,
                      keepdims=True)
        s = p1 + p2 + 1e-8
        w0_ref[pl.ds(t * TILE, TILE), :] = p1 / s
        w1_ref[pl.ds(t * TILE, TILE), :] = p2 / s
        oh1_s[pl.ds(t * TILE, TILE), :] = (lane == id1).astype(jnp.float32)
        oh2_s[pl.ds(t * TILE, TILE), :] = (lane == id2).astype(jnp.float32)

    @pl.when(v < 2 * NUM_TILES)
    def _prefix():
        base = jnp.where(v < NUM_TILES, 0, SEQ)
        oh = jnp.where(v < NUM_TILES,
                       oh1_s[pl.ds(t * TILE, TILE), :],
                       oh2_s[pl.ds(t * TILE, TILE), :])
        row = jax.lax.broadcasted_iota(jnp.int32, (TILE, TILE), 0)
        col = jax.lax.broadcasted_iota(jnp.int32, (TILE, TILE), 1)
        ltri = (row > col).astype(jnp.float32)
        rank = jnp.dot(ltri, oh, preferred_element_type=jnp.float32)
        snap = run_s[...]
        pig = jnp.sum(oh * (rank + snap), axis=-1, keepdims=True)
        pos_ref[pl.ds(base + t * TILE, TILE), :] = pig.astype(jnp.int32)
        run_s[...] = snap + jnp.sum(oh, axis=0, keepdims=True)

    @pl.when(v == 2 * NUM_TILES - 1)
    def _plan_offsets():
        total = run_s[...]
        padded = jnp.floor((total + (TG - 1)) * (1.0 / TG)) * TG
        erow = jax.lax.broadcasted_iota(jnp.int32, (NUM_EXPERTS,
                                                    NUM_EXPERTS), 0)
        ecol = jax.lax.broadcasted_iota(jnp.int32, (NUM_EXPERTS,
                                                    NUM_EXPERTS), 1)
        utri = (erow < ecol).astype(jnp.float32)
        offs = jnp.dot(padded, utri, preferred_element_type=jnp.float32)
        ends = offs + padded
        off_s[...] = offs
        jiota = jax.lax.broadcasted_iota(jnp.int32, (1, 32), 1) * TG
        te = jnp.zeros((1, 32), jnp.int32)
        for e in range(NUM_EXPERTS):
            ende = ends[0:1, e:e + 1].astype(jnp.int32)
            te = te + jnp.where(jiota >= ende, 1, 0)
        te_ref[...] = jnp.minimum(te, NUM_EXPERTS - 1)

    @pl.when(v >= 2 * NUM_TILES)
    def _add_offsets():
        u = v - 2 * NUM_TILES
        t2 = u % NUM_TILES
        base = jnp.where(u < NUM_TILES, 0, SEQ)
        oh = jnp.where(u < NUM_TILES,
                       oh1_s[pl.ds(t2 * TILE, TILE), :],
                       oh2_s[pl.ds(t2 * TILE, TILE), :])
        off_row = jnp.sum(oh * off_s[...], axis=-1,
                          keepdims=True).astype(jnp.int32)
        pos_ref[pl.ds(base + t2 * TILE, TILE), :] = (
            pos_ref[pl.ds(base + t2 * TILE, TILE), :] + off_row)


def _dispatch_kernel(pos_hbm, x_hbm, xs_hbm, posw_v, tokw_v, rows_v, sem):
    wid = lax.axis_index("s")
    abase = wid * CHUNK
    lanes = lax.iota(jnp.int32, 16)

    for b in range(2):
        pltpu.sync_copy(
            pos_hbm.at[pl.ds(abase + b * (CHUNK // 2), CHUNK // 2)],
            posw_v.at[b])
        for k in range(NV // 2):
            tokw_v[b, pl.ds(k * 16, 16)] = (
                abase + (b * (NV // 2) + k) * 16 + lanes) % SEQ

    for b in range(2):
        pltpu.async_copy(x_hbm.at[tokw_v.at[b]], rows_v, sem).wait()
        pltpu.async_copy(rows_v, xs_hbm.at[posw_v.at[b]], sem).wait()


def _unpermute_kernel(pos_hbm, ys_hbm, z_hbm, posf_v, rows_v, sem):
    wid = lax.axis_index("s")
    abase = wid * CHUNK
    pltpu.sync_copy(pos_hbm.at[pl.ds(abase, CHUNK)], posf_v)
    for b in range(2):
        pltpu.async_copy(ys_hbm.at[posf_v.at[pl.ds(b * (CHUNK // 2),
                                                   CHUNK // 2)]],
                         rows_v, sem).wait()
        pltpu.sync_copy(rows_v, z_hbm.at[pl.ds(abase + b * (CHUNK // 2),
                                               CHUNK // 2)])


def _grouped_kernel(s_ref, xs_ref, w1_ref, b1_ref, w2_ref, b2_ref, ys_ref):
    h = jnp.dot(xs_ref[...], w1_ref[0], preferred_element_type=jnp.float32)
    h = _gelu(h + b1_ref[0])
    y = jnp.dot(h, w2_ref[0], preferred_element_type=jnp.float32)
    ys_ref[...] = y + b2_ref[0]


def _resident(shape):
    return pl.BlockSpec(shape, lambda *_: tuple(0 for _ in shape))


def _per_g(shape):
    return pl.BlockSpec(shape, lambda g, t: (g,) + tuple(0 for _ in shape[1:]))


@jax.jit
def kernel(x, router_W, router_b, expert_W1, expert_b1, expert_W2, expert_b2,
           micro_router_W, micro_router_b, micro_W1, micro_b1, micro_W2,
           micro_b2, micro_ln_g, micro_ln_b, norm_g, norm_b):
    B, S, D = x.shape
    xf = x.reshape(S, D)

    pos, w0, w1, tile_e = pl.pallas_call(
        _plan_kernel,
        grid=(4 * NUM_TILES,),
        in_specs=[
            _resident((S, D)),
            _resident((D, NUM_EXPERTS)),
            _resident((1, NUM_EXPERTS)),
        ],
        out_specs=[
            _resident((2 * S, 1)),
            _resident((S, 1)),
            _resident((S, 1)),
            _resident((1, 32)),
        ],
        out_shape=[jax.ShapeDtypeStruct((2 * S, 1), jnp.int32),
                   jax.ShapeDtypeStruct((S, 1), jnp.float32),
                   jax.ShapeDtypeStruct((S, 1), jnp.float32),
                   jax.ShapeDtypeStruct((1, 32), jnp.int32)],
        scratch_shapes=[pltpu.VMEM((S, NUM_EXPERTS), jnp.float32),
                        pltpu.VMEM((S, NUM_EXPERTS), jnp.float32),
                        pltpu.VMEM((1, NUM_EXPERTS), jnp.float32),
                        pltpu.VMEM((1, NUM_EXPERTS), jnp.float32)],
        compiler_params=pltpu.CompilerParams(
            dimension_semantics=("arbitrary",),
        ),
    )(xf, router_W, router_b.reshape(1, -1))

    pos_f = pos.reshape(2 * S)

    mesh = plsc.VectorSubcoreMesh(core_axis_name="c", subcore_axis_name="s",
                                  num_cores=1)
    xs = pl.kernel(
        _dispatch_kernel,
        mesh=mesh,
        out_type=jax.ShapeDtypeStruct((NP, D), jnp.float32),
        scratch_types=[pltpu.VMEM((2, CHUNK // 2), jnp.int32),
                       pltpu.VMEM((2, CHUNK // 2), jnp.int32),
                       pltpu.VMEM((CHUNK // 2, D), jnp.float32),
                       pltpu.SemaphoreType.DMA],
    )(pos_f, xf)

    ys = pl.pallas_call(
        _grouped_kernel,
        grid_spec=pltpu.PrefetchScalarGridSpec(
            num_scalar_prefetch=1,
            grid=(NT,),
            in_specs=[
                pl.BlockSpec((TG, D), lambda i, s: (i, 0)),
                pl.BlockSpec((1, DIM, EXPERT_DIM), lambda i, s: (s[i], 0, 0)),
                pl.BlockSpec((1, 1, EXPERT_DIM), lambda i, s: (s[i], 0, 0)),
                pl.BlockSpec((1, EXPERT_DIM, DIM), lambda i, s: (s[i], 0, 0)),
                pl.BlockSpec((1, 1, DIM), lambda i, s: (s[i], 0, 0)),
            ],
            out_specs=pl.BlockSpec((TG, D), lambda i, s: (i, 0)),
        ),
        out_shape=jax.ShapeDtypeStruct((NP, D), jnp.float32),
        compiler_params=pltpu.CompilerParams(
            dimension_semantics=("arbitrary",),
        ),
    )(tile_e.reshape(32), xs, expert_W1,
      expert_b1.reshape(NUM_EXPERTS, 1, EXPERT_DIM),
      expert_W2, expert_b2.reshape(NUM_EXPERTS, 1, DIM))

    z = pl.kernel(
        _unpermute_kernel,
        mesh=mesh,
        out_type=jax.ShapeDtypeStruct((2 * S, D), jnp.float32),
        scratch_types=[pltpu.VMEM((CHUNK,), jnp.int32),
                       pltpu.VMEM((CHUNK // 2, D), jnp.float32),
                       pltpu.SemaphoreType.DMA],
    )(pos_f, ys)

    out = pl.pallas_call(
        _micro_kernel,
        grid=(M_STEPS, NUM_TILES),
        in_specs=[
            _resident((2 * S, D)),
            _resident((S, 1)),
            _resident((S, 1)),
            _resident((D, NUM_MICROS)),
            _resident((1, NUM_MICROS)),
            _per_g((1, MG, DIM, MICRO_HID)),
            _per_g((1, MG, 1, MICRO_HID)),
            _per_g((1, MG, MICRO_HID, DIM)),
            _per_g((1, MG, DIM)),
            _per_g((1, MG, DIM)),
            _per_g((1, MG, DIM)),
            _resident((1, DIM)),
            _resident((1, DIM)),
        ],
        out_specs=_resident((S, D)),
        out_shape=jax.ShapeDtypeStruct((S, D), jnp.float32),
        scratch_shapes=[pltpu.VMEM((S, D), jnp.float32),
                        pltpu.VMEM((S, NUM_MICROS), jnp.float32)],
        compiler_params=pltpu.CompilerParams(
            dimension_semantics=("arbitrary", "arbitrary"),
        ),
    )(z, w0, w1, micro_router_W,
      micro_router_b.reshape(1, -1),
      micro_W1.reshape(M_STEPS, MG, DIM, MICRO_HID),
      micro_b1.reshape(M_STEPS, MG, 1, MICRO_HID),
      micro_W2.reshape(M_STEPS, MG, MICRO_HID, DIM),
      micro_b2.reshape(M_STEPS, MG, DIM),
      micro_ln_g.reshape(M_STEPS, MG, DIM),
      micro_ln_b.reshape(M_STEPS, MG, DIM),
      norm_g.reshape(1, -1), norm_b.reshape(1, -1))

    return out.reshape(B, S, D)


# speedup vs baseline: 1.1587x; 1.0436x over previous
import jax
import jax.numpy as jnp
from jax import lax
from jax.experimental import pallas as pl
from jax.experimental.pallas import tpu as pltpu
from jax.experimental.pallas import tpu_sc as plsc

DIM = 768
NUM_EXPERTS = 8
NUM_MICROS = 16
TOP_K = 2
TOP_K_MICROS = 8
EXPERT_DIM = 1536
MICRO_HID = DIM // 2
SEQ = 2048
TILE = 512
NUM_TILES = SEQ // TILE
EG = 2
MG = 4
E_STEPS = NUM_EXPERTS // EG
M_STEPS = NUM_MICROS // MG
EPS = 1e-5


def _gelu(v):
    return 0.5 * v * (1.0 + jax.lax.erf(v * 0.7071067811865476))


def _layer_norm(v, g, b):
    mu = jnp.mean(v, axis=-1, keepdims=True)
    var = jnp.mean((v - mu) ** 2, axis=-1, keepdims=True)
    return (v - mu) * jax.lax.rsqrt(var + EPS) * g + b


def _topk_mask_combine(probs, k):
    work = probs
    thr = None
    sel_sum = jnp.zeros(probs.shape[:-1] + (1,), probs.dtype)
    for _ in range(k):
        thr = jnp.max(work, axis=-1, keepdims=True)
        sel_sum = sel_sum + thr
        work = jnp.where(work >= thr, -jnp.inf, work)
    mask = probs >= thr
    return jnp.where(mask, probs, 0.0) / (sel_sum + 1e-8)


def _col(combine, idx):
    lane = jax.lax.broadcasted_iota(jnp.int32, combine.shape, 1)
    return jnp.sum(jnp.where(lane == idx, combine, 0.0), axis=-1,
                   keepdims=True)


def _expert_kernel(x_ref, rw_ref, rb_ref, w1_ref, b1_ref, w2_ref,
                   b2_ref, out_ref, cmb_ref):
    g = pl.program_id(0)
    t = pl.program_id(1)
    xt = x_ref[pl.ds(t * TILE, TILE), :]

    @pl.when(g == 0)
    def _router():
        logits = jnp.dot(xt, rw_ref[...], preferred_element_type=jnp.float32)
        logits = logits + rb_ref[...]
        probs = jax.nn.softmax(logits, axis=-1)
        cmb_ref[pl.ds(t * TILE, TILE), :] = _topk_mask_combine(probs, TOP_K)

    combine = cmb_ref[pl.ds(t * TILE, TILE), :]

    acc = None
    for j in range(EG):
        h = jnp.dot(xt, w1_ref[0, j], preferred_element_type=jnp.float32)
        h = _gelu(h + b1_ref[0, j])
        eo = jnp.dot(h, w2_ref[0, j], preferred_element_type=jnp.float32)
        eo = (eo + b2_ref[0, j]) * _col(combine, g * EG + j)
        acc = eo if acc is None else acc + eo

    @pl.when(g == 0)
    def _init():
        out_ref[pl.ds(t * TILE, TILE), :] = acc

    @pl.when(g > 0)
    def _acc():
        out_ref[pl.ds(t * TILE, TILE), :] += acc


def _micro_kernel(z_ref, w0_ref, w1s_ref, rw_ref, rb_ref, w1_ref, b1_ref,
                  w2_ref, b2_ref, lng_ref, lnb_ref, ng_ref, nb_ref, out_ref,
                  eo_s, cmb_ref):
    g = pl.program_id(0)
    t = pl.program_id(1)

    @pl.when(g == 0)
    def _combine_experts():
        zt0 = z_ref[pl.ds(t * TILE, TILE), :]
        zt1 = z_ref[pl.ds(SEQ + t * TILE, TILE), :]
        w0 = w0_ref[pl.ds(t * TILE, TILE), :]
        w1 = w1s_ref[pl.ds(t * TILE, TILE), :]
        eo_s[pl.ds(t * TILE, TILE), :] = zt0 * w0 + zt1 * w1

    xt = eo_s[pl.ds(t * TILE, TILE), :]

    @pl.when(g == 0)
    def _router():
        logits = jnp.dot(xt, rw_ref[...], preferred_element_type=jnp.float32)
        logits = logits + rb_ref[...]
        probs = jax.nn.softmax(logits, axis=-1)
        cmb_ref[pl.ds(t * TILE, TILE), :] = _topk_mask_combine(
            probs, TOP_K_MICROS)

    mcombine = cmb_ref[pl.ds(t * TILE, TILE), :]

    acc = None
    for j in range(MG):
        mh = jnp.dot(xt, w1_ref[0, j], preferred_element_type=jnp.float32)
        mh = _gelu(mh + b1_ref[0, j])
        mf = jnp.dot(mh, w2_ref[0, j], preferred_element_type=jnp.float32)
        pre = xt + mf + b2_ref[0, j]
        mo = _layer_norm(pre, lng_ref[0, j], lnb_ref[0, j])
        mo = mo * _col(mcombine, g * MG + j)
        acc = mo if acc is None else acc + mo

    @pl.when(g == 0)
    def _init():
        out_ref[pl.ds(t * TILE, TILE), :] = acc

    @pl.when(g > 0)
    def _acc():
        out_ref[pl.ds(t * TILE, TILE), :] += acc

    @pl.when(g == M_STEPS - 1)
    def _final():
        combined = xt + 0.1 * out_ref[pl.ds(t * TILE, TILE), :]
        out_ref[pl.ds(t * TILE, TILE), :] = _layer_norm(
            combined, ng_ref[...], nb_ref[...])


TG = 256
NP = 2 * SEQ + NUM_EXPERTS * TG
NT = NP // TG
NW = 32
CHUNK = 2 * SEQ // NW
NV = CHUNK // 16


def _plan_kernel(x_ref, rw_ref, rb_ref, pos_ref, w0_ref, w1_ref, te_ref,
                 oh1_s, oh2_s, run_s, off_s):
    v = pl.program_id(0)
    t = v % NUM_TILES

    @pl.when(v == 0)
    def _init():
        run_s[...] = jnp.zeros((1, NUM_EXPERTS), jnp.float32)

    @pl.when(v < NUM_TILES)
    def _router():
        xt = x_ref[pl.ds(t * TILE, TILE), :]
        logits = jnp.dot(xt, rw_ref[...], preferred_element_type=jnp.float32)
        logits = logits + rb_ref[...]
        probs = jax.nn.softmax(logits, axis=-1)
        lane = jax.lax.broadcasted_iota(jnp.int32, probs.shape, 1)
        p1 = jnp.max(probs, axis=-1, keepdims=True)
        id1 = jnp.min(jnp.where(probs >= p1, lane, NUM_EXPERTS), axis=-1,
                      keepdims=True)
        masked = jnp.where(lane == id1, -jnp.inf, probs)
        p2 = jnp.max(masked, axis=-1, keepdims=True)
        id2 = jnp.min(jnp.where(masked >= p2, lane, NUM_EXPERTS), axis=-1,
                      keepdims=True)
        s = p1 + p2 + 1e-8
        w0_ref[pl.ds(t * TILE, TILE), :] = p1 / s
        w1_ref[pl.ds(t * TILE, TILE), :] = p2 / s
        oh1_s[pl.ds(t * TILE, TILE), :] = (lane == id1).astype(jnp.float32)
        oh2_s[pl.ds(t * TILE, TILE), :] = (lane == id2).astype(jnp.float32)

    @pl.when(v < 2 * NUM_TILES)
    def _prefix():
        base = jnp.where(v < NUM_TILES, 0, SEQ)
        oh = jnp.where(v < NUM_TILES,
                       oh1_s[pl.ds(t * TILE, TILE), :],
                       oh2_s[pl.ds(t * TILE, TILE), :])
        row = jax.lax.broadcasted_iota(jnp.int32, (TILE, TILE), 0)
        col = jax.lax.broadcasted_iota(jnp.int32, (TILE, TILE), 1)
        ltri = (row > col).astype(jnp.float32)
        rank = jnp.dot(ltri, oh, preferred_element_type=jnp.float32)
        snap = run_s[...]
        pig = jnp.sum(oh * (rank + snap), axis=-1, keepdims=True)
        pos_ref[pl.ds(base + t * TILE, TILE), :] = pig.astype(jnp.int32)
        run_s[...] = snap + jnp.sum(oh, axis=0, keepdims=True)

    @pl.when(v == 2 * NUM_TILES - 1)
    def _plan_offsets():
        total = run_s[...]
        padded = jnp.floor((total + (TG - 1)) * (1.0 / TG)) * TG
        erow = jax.lax.broadcasted_iota(jnp.int32, (NUM_EXPERTS,
                                                    NUM_EXPERTS), 0)
        ecol = jax.lax.broadcasted_iota(jnp.int32, (NUM_EXPERTS,
                                                    NUM_EXPERTS), 1)
        utri = (erow < ecol).astype(jnp.float32)
        offs = jnp.dot(padded, utri, preferred_element_type=jnp.float32)
        ends = offs + padded
        off_s[...] = offs
        jiota = jax.lax.broadcasted_iota(jnp.int32, (1, 32), 1) * TG
        te = jnp.zeros((1, 32), jnp.int32)
        for e in range(NUM_EXPERTS):
            ende = ends[0:1, e:e + 1].astype(jnp.int32)
            te = te + jnp.where(jiota >= ende, 1, 0)
        te_ref[...] = jnp.minimum(te, NUM_EXPERTS - 1)

    @pl.when(v >= 2 * NUM_TILES)
    def _add_offsets():
        u = v - 2 * NUM_TILES
        t2 = u % NUM_TILES
        base = jnp.where(u < NUM_TILES, 0, SEQ)
        oh = jnp.where(u < NUM_TILES,
                       oh1_s[pl.ds(t2 * TILE, TILE), :],
                       oh2_s[pl.ds(t2 * TILE, TILE), :])
        off_row = jnp.sum(oh * off_s[...], axis=-1,
                          keepdims=True).astype(jnp.int32)
        pos_ref[pl.ds(base + t2 * TILE, TILE), :] = (
            pos_ref[pl.ds(base + t2 * TILE, TILE), :] + off_row)


def _dispatch_kernel(pos_hbm, x_hbm, xs_hbm, posw_v, tokw_v, rows_v, sem):
    wid = lax.axis_index("s") * 2 + lax.axis_index("c")
    abase = wid * CHUNK
    lanes = lax.iota(jnp.int32, 16)

    for b in range(2):
        pltpu.sync_copy(
            pos_hbm.at[pl.ds(abase + b * (CHUNK // 2), CHUNK // 2)],
            posw_v.at[b])
        for k in range(NV // 2):
            tokw_v[b, pl.ds(k * 16, 16)] = (
                abase + (b * (NV // 2) + k) * 16 + lanes) % SEQ

    for b in range(2):
        pltpu.async_copy(x_hbm.at[tokw_v.at[b]], rows_v, sem).wait()
        pltpu.async_copy(rows_v, xs_hbm.at[posw_v.at[b]], sem).wait()


def _unpermute_kernel(pos_hbm, ys_hbm, z_hbm, posf_v, rows_v, sem):
    wid = lax.axis_index("s") * 2 + lax.axis_index("c")
    abase = wid * CHUNK
    pltpu.sync_copy(pos_hbm.at[pl.ds(abase, CHUNK)], posf_v)
    for b in range(2):
        pltpu.async_copy(ys_hbm.at[posf_v.at[pl.ds(b * (CHUNK // 2),
                                                   CHUNK // 2)]],
                         rows_v, sem).wait()
        pltpu.sync_copy(rows_v, z_hbm.at[pl.ds(abase + b * (CHUNK // 2),
                                               CHUNK // 2)])


def _grouped_kernel(s_ref, xs_ref, w1_ref, b1_ref, w2_ref, b2_ref, ys_ref):
    h = jnp.dot(xs_ref[...], w1_ref[0], preferred_element_type=jnp.float32)
    h = _gelu(h + b1_ref[0])
    y = jnp.dot(h, w2_ref[0], preferred_element_type=jnp.float32)
    ys_ref[...] = y + b2_ref[0]


def _resident(shape):
    return pl.BlockSpec(shape, lambda *_: tuple(0 for _ in shape))


def _per_g(shape):
    return pl.BlockSpec(shape, lambda g, t: (g,) + tuple(0 for _ in shape[1:]))


@jax.jit
def kernel(x, router_W, router_b, expert_W1, expert_b1, expert_W2, expert_b2,
           micro_router_W, micro_router_b, micro_W1, micro_b1, micro_W2,
           micro_b2, micro_ln_g, micro_ln_b, norm_g, norm_b):
    B, S, D = x.shape
    xf = x.reshape(S, D)

    pos, w0, w1, tile_e = pl.pallas_call(
        _plan_kernel,
        grid=(4 * NUM_TILES,),
        in_specs=[
            _resident((S, D)),
            _resident((D, NUM_EXPERTS)),
            _resident((1, NUM_EXPERTS)),
        ],
        out_specs=[
            _resident((2 * S, 1)),
            _resident((S, 1)),
            _resident((S, 1)),
            _resident((1, 32)),
        ],
        out_shape=[jax.ShapeDtypeStruct((2 * S, 1), jnp.int32),
                   jax.ShapeDtypeStruct((S, 1), jnp.float32),
                   jax.ShapeDtypeStruct((S, 1), jnp.float32),
                   jax.ShapeDtypeStruct((1, 32), jnp.int32)],
        scratch_shapes=[pltpu.VMEM((S, NUM_EXPERTS), jnp.float32),
                        pltpu.VMEM((S, NUM_EXPERTS), jnp.float32),
                        pltpu.VMEM((1, NUM_EXPERTS), jnp.float32),
                        pltpu.VMEM((1, NUM_EXPERTS), jnp.float32)],
        compiler_params=pltpu.CompilerParams(
            dimension_semantics=("arbitrary",),
        ),
    )(xf, router_W, router_b.reshape(1, -1))

    pos_f = pos.reshape(2 * S)

    mesh = plsc.VectorSubcoreMesh(core_axis_name="c", subcore_axis_name="s",
                                  num_cores=2)
    xs = pl.kernel(
        _dispatch_kernel,
        mesh=mesh,
        out_type=jax.ShapeDtypeStruct((NP, D), jnp.float32),
        scratch_types=[pltpu.VMEM((2, CHUNK // 2), jnp.int32),
                       pltpu.VMEM((2, CHUNK // 2), jnp.int32),
                       pltpu.VMEM((CHUNK // 2, D), jnp.float32),
                       pltpu.SemaphoreType.DMA],
    )(pos_f, xf)

    ys = pl.pallas_call(
        _grouped_kernel,
        grid_spec=pltpu.PrefetchScalarGridSpec(
            num_scalar_prefetch=1,
            grid=(NT,),
            in_specs=[
                pl.BlockSpec((TG, D), lambda i, s: (i, 0)),
                pl.BlockSpec((1, DIM, EXPERT_DIM), lambda i, s: (s[i], 0, 0)),
                pl.BlockSpec((1, 1, EXPERT_DIM), lambda i, s: (s[i], 0, 0)),
                pl.BlockSpec((1, EXPERT_DIM, DIM), lambda i, s: (s[i], 0, 0)),
                pl.BlockSpec((1, 1, DIM), lambda i, s: (s[i], 0, 0)),
            ],
            out_specs=pl.BlockSpec((TG, D), lambda i, s: (i, 0)),
        ),
        out_shape=jax.ShapeDtypeStruct((NP, D), jnp.float32),
        compiler_params=pltpu.CompilerParams(
            dimension_semantics=("arbitrary",),
        ),
    )(tile_e.reshape(32), xs, expert_W1,
      expert_b1.reshape(NUM_EXPERTS, 1, EXPERT_DIM),
      expert_W2, expert_b2.reshape(NUM_EXPERTS, 1, DIM))

    z = pl.kernel(
        _unpermute_kernel,
        mesh=mesh,
        out_type=jax.ShapeDtypeStruct((2 * S, D), jnp.float32),
        scratch_types=[pltpu.VMEM((CHUNK,), jnp.int32),
                       pltpu.VMEM((CHUNK // 2, D), jnp.float32),
                       pltpu.SemaphoreType.DMA],
    )(pos_f, ys)

    out = pl.pallas_call(
        _micro_kernel,
        grid=(M_STEPS, NUM_TILES),
        in_specs=[
            _resident((2 * S, D)),
            _resident((S, 1)),
            _resident((S, 1)),
            _resident((D, NUM_MICROS)),
            _resident((1, NUM_MICROS)),
            _per_g((1, MG, DIM, MICRO_HID)),
            _per_g((1, MG, 1, MICRO_HID)),
            _per_g((1, MG, MICRO_HID, DIM)),
            _per_g((1, MG, DIM)),
            _per_g((1, MG, DIM)),
            _per_g((1, MG, DIM)),
            _resident((1, DIM)),
            _resident((1, DIM)),
        ],
        out_specs=_resident((S, D)),
        out_shape=jax.ShapeDtypeStruct((S, D), jnp.float32),
        scratch_shapes=[pltpu.VMEM((S, D), jnp.float32),
                        pltpu.VMEM((S, NUM_MICROS), jnp.float32)],
        compiler_params=pltpu.CompilerParams(
            dimension_semantics=("arbitrary", "arbitrary"),
        ),
    )(z, w0, w1, micro_router_W,
      micro_router_b.reshape(1, -1),
      micro_W1.reshape(M_STEPS, MG, DIM, MICRO_HID),
      micro_b1.reshape(M_STEPS, MG, 1, MICRO_HID),
      micro_W2.reshape(M_STEPS, MG, MICRO_HID, DIM),
      micro_b2.reshape(M_STEPS, MG, DIM),
      micro_ln_g.reshape(M_STEPS, MG, DIM),
      micro_ln_b.reshape(M_STEPS, MG, DIM),
      norm_g.reshape(1, -1), norm_b.reshape(1, -1))

    return out.reshape(B, S, D)


# speedup vs baseline: 1.1597x; 1.0009x over previous
import jax
import jax.numpy as jnp
from jax import lax
from jax.experimental import pallas as pl
from jax.experimental.pallas import tpu as pltpu
from jax.experimental.pallas import tpu_sc as plsc

DIM = 768
NUM_EXPERTS = 8
NUM_MICROS = 16
TOP_K = 2
TOP_K_MICROS = 8
EXPERT_DIM = 1536
MICRO_HID = DIM // 2
SEQ = 2048
TILE = 512
NUM_TILES = SEQ // TILE
MG = 4
M_STEPS = NUM_MICROS // MG
EPS = 1e-5


def _gelu(v):
    return 0.5 * v * (1.0 + jax.lax.erf(v * 0.7071067811865476))


def _layer_norm(v, g, b):
    mu = jnp.mean(v, axis=-1, keepdims=True)
    var = jnp.mean((v - mu) ** 2, axis=-1, keepdims=True)
    return (v - mu) * jax.lax.rsqrt(var + EPS) * g + b


def _topk_mask_combine(probs, k):
    work = probs
    thr = None
    sel_sum = jnp.zeros(probs.shape[:-1] + (1,), probs.dtype)
    for _ in range(k):
        thr = jnp.max(work, axis=-1, keepdims=True)
        sel_sum = sel_sum + thr
        work = jnp.where(work >= thr, -jnp.inf, work)
    mask = probs >= thr
    return jnp.where(mask, probs, 0.0) / (sel_sum + 1e-8)


def _col(combine, idx):
    lane = jax.lax.broadcasted_iota(jnp.int32, combine.shape, 1)
    return jnp.sum(jnp.where(lane == idx, combine, 0.0), axis=-1,
                   keepdims=True)


def _micro_kernel(z_ref, w0_ref, w1s_ref, rw_ref, rb_ref, w1_ref, b1_ref,
                  w2_ref, b2_ref, lng_ref, lnb_ref, ng_ref, nb_ref, out_ref,
                  eo_s, cmb_ref):
    g = pl.program_id(0)
    t = pl.program_id(1)

    @pl.when(g == 0)
    def _combine_experts():
        zt0 = z_ref[pl.ds(t * TILE, TILE), :]
        zt1 = z_ref[pl.ds(SEQ + t * TILE, TILE), :]
        w0 = w0_ref[pl.ds(t * TILE, TILE), :]
        w1 = w1s_ref[pl.ds(t * TILE, TILE), :]
        eo_s[pl.ds(t * TILE, TILE), :] = zt0 * w0 + zt1 * w1

    xt = eo_s[pl.ds(t * TILE, TILE), :]

    @pl.when(g == 0)
    def _router():
        logits = jnp.dot(xt, rw_ref[...], preferred_element_type=jnp.float32)
        logits = logits + rb_ref[...]
        probs = jax.nn.softmax(logits, axis=-1)
        cmb_ref[pl.ds(t * TILE, TILE), :] = _topk_mask_combine(
            probs, TOP_K_MICROS)

    mcombine = cmb_ref[pl.ds(t * TILE, TILE), :]

    acc = None
    for j in range(MG):
        mh = jnp.dot(xt, w1_ref[0, j], preferred_element_type=jnp.float32)
        mh = _gelu(mh + b1_ref[0, j])
        mf = jnp.dot(mh, w2_ref[0, j], preferred_element_type=jnp.float32)
        pre = xt + mf + b2_ref[0, j]
        mo = _layer_norm(pre, lng_ref[0, j], lnb_ref[0, j])
        mo = mo * _col(mcombine, g * MG + j)
        acc = mo if acc is None else acc + mo

    @pl.when(g == 0)
    def _init():
        out_ref[pl.ds(t * TILE, TILE), :] = acc

    @pl.when(g > 0)
    def _acc():
        out_ref[pl.ds(t * TILE, TILE), :] += acc

    @pl.when(g == M_STEPS - 1)
    def _final():
        combined = xt + 0.1 * out_ref[pl.ds(t * TILE, TILE), :]
        out_ref[pl.ds(t * TILE, TILE), :] = _layer_norm(
            combined, ng_ref[...], nb_ref[...])


TG = 256
NP = 2 * SEQ + NUM_EXPERTS * TG
NT = NP // TG
NW = 32
CHUNK = 2 * SEQ // NW
NV = CHUNK // 16


def _plan_kernel(x_ref, rw_ref, rb_ref, pos_ref, w0_ref, w1_ref, te_ref,
                 oh1_s, oh2_s, run_s, off_s):
    v = pl.program_id(0)
    t = v % NUM_TILES

    @pl.when(v == 0)
    def _init():
        run_s[...] = jnp.zeros((1, NUM_EXPERTS), jnp.float32)

    @pl.when(v < NUM_TILES)
    def _router():
        xt = x_ref[pl.ds(t * TILE, TILE), :]
        logits = jnp.dot(xt, rw_ref[...], preferred_element_type=jnp.float32)
        logits = logits + rb_ref[...]
        probs = jax.nn.softmax(logits, axis=-1)
        lane = jax.lax.broadcasted_iota(jnp.int32, probs.shape, 1)
        p1 = jnp.max(probs, axis=-1, keepdims=True)
        id1 = jnp.min(jnp.where(probs >= p1, lane, NUM_EXPERTS), axis=-1,
                      keepdims=True)
        masked = jnp.where(lane == id1, -jnp.inf, probs)
        p2 = jnp.max(masked, axis=-1, keepdims=True)
        id2 = jnp.min(jnp.where(masked >= p2, lane, NUM_EXPERTS), axis=-1,
                      keepdims=True)
        s = p1 + p2 + 1e-8
        w0_ref[pl.ds(t * TILE, TILE), :] = p1 / s
        w1_ref[pl.ds(t * TILE, TILE), :] = p2 / s
        oh1_s[pl.ds(t * TILE, TILE), :] = (lane == id1).astype(jnp.float32)
        oh2_s[pl.ds(t * TILE, TILE), :] = (lane == id2).astype(jnp.float32)

    @pl.when(v < 2 * NUM_TILES)
    def _prefix():
        base = jnp.where(v < NUM_TILES, 0, SEQ)
        oh = jnp.where(v < NUM_TILES,
                       oh1_s[pl.ds(t * TILE, TILE), :],
                       oh2_s[pl.ds(t * TILE, TILE), :])
        row = jax.lax.broadcasted_iota(jnp.int32, (TILE, TILE), 0)
        col = jax.lax.broadcasted_iota(jnp.int32, (TILE, TILE), 1)
        ltri = (row > col).astype(jnp.float32)
        rank = jnp.dot(ltri, oh, preferred_element_type=jnp.float32)
        snap = run_s[...]
        pig = jnp.sum(oh * (rank + snap), axis=-1, keepdims=True)
        pos_ref[pl.ds(base + t * TILE, TILE), :] = pig.astype(jnp.int32)
        run_s[...] = snap + jnp.sum(oh, axis=0, keepdims=True)

    @pl.when(v == 2 * NUM_TILES - 1)
    def _plan_offsets():
        total = run_s[...]
        padded = jnp.floor((total + (TG - 1)) * (1.0 / TG)) * TG
        erow = jax.lax.broadcasted_iota(jnp.int32, (NUM_EXPERTS,
                                                    NUM_EXPERTS), 0)
        ecol = jax.lax.broadcasted_iota(jnp.int32, (NUM_EXPERTS,
                                                    NUM_EXPERTS), 1)
        utri = (erow < ecol).astype(jnp.float32)
        offs = jnp.dot(padded, utri, preferred_element_type=jnp.float32)
        ends = offs + padded
        off_s[...] = offs
        jiota = jax.lax.broadcasted_iota(jnp.int32, (1, 32), 1) * TG
        te = jnp.zeros((1, 32), jnp.int32)
        for e in range(NUM_EXPERTS):
            ende = ends[0:1, e:e + 1].astype(jnp.int32)
            te = te + jnp.where(jiota >= ende, 1, 0)
        te_ref[...] = jnp.minimum(te, NUM_EXPERTS - 1)

    @pl.when(v >= 2 * NUM_TILES)
    def _add_offsets():
        u = v - 2 * NUM_TILES
        t2 = u % NUM_TILES
        base = jnp.where(u < NUM_TILES, 0, SEQ)
        oh = jnp.where(u < NUM_TILES,
                       oh1_s[pl.ds(t2 * TILE, TILE), :],
                       oh2_s[pl.ds(t2 * TILE, TILE), :])
        off_row = jnp.sum(oh * off_s[...], axis=-1,
                          keepdims=True).astype(jnp.int32)
        pos_ref[pl.ds(base + t2 * TILE, TILE), :] = (
            pos_ref[pl.ds(base + t2 * TILE, TILE), :] + off_row)


def _dispatch_kernel(pos_hbm, x_hbm, xs_hbm, posw_v, tokw_v, rows_v, sem):
    wid = lax.axis_index("s") * 2 + lax.axis_index("c")
    abase = wid * CHUNK
    lanes = lax.iota(jnp.int32, 16)

    for b in range(2):
        pltpu.sync_copy(
            pos_hbm.at[pl.ds(abase + b * (CHUNK // 2), CHUNK // 2)],
            posw_v.at[b])
        for k in range(NV // 2):
            tokw_v[b, pl.ds(k * 16, 16)] = (
                abase + (b * (NV // 2) + k) * 16 + lanes) % SEQ

    for b in range(2):
        pltpu.async_copy(x_hbm.at[tokw_v.at[b]], rows_v, sem).wait()
        pltpu.async_copy(rows_v, xs_hbm.at[posw_v.at[b]], sem).wait()


def _unpermute_kernel(pos_hbm, ys_hbm, z_hbm, posf_v, rows_v, sem):
    wid = lax.axis_index("s") * 2 + lax.axis_index("c")
    abase = wid * CHUNK
    pltpu.sync_copy(pos_hbm.at[pl.ds(abase, CHUNK)], posf_v)
    for b in range(2):
        pltpu.async_copy(ys_hbm.at[posf_v.at[pl.ds(b * (CHUNK // 2),
                                                   CHUNK // 2)]],
                         rows_v, sem).wait()
        pltpu.sync_copy(rows_v, z_hbm.at[pl.ds(abase + b * (CHUNK // 2),
                                               CHUNK // 2)])


def _grouped_kernel(s_ref, xs_ref, w1_ref, b1_ref, w2_ref, b2_ref, ys_ref):
    h = jnp.dot(xs_ref[...], w1_ref[0], preferred_element_type=jnp.float32)
    h = _gelu(h + b1_ref[0])
    y = jnp.dot(h, w2_ref[0], preferred_element_type=jnp.float32)
    ys_ref[...] = y + b2_ref[0]


def _resident(shape):
    return pl.BlockSpec(shape, lambda *_: tuple(0 for _ in shape))


def _per_g(shape):
    return pl.BlockSpec(shape, lambda g, t: (g,) + tuple(0 for _ in shape[1:]))


@jax.jit
def kernel(x, router_W, router_b, expert_W1, expert_b1, expert_W2, expert_b2,
           micro_router_W, micro_router_b, micro_W1, micro_b1, micro_W2,
           micro_b2, micro_ln_g, micro_ln_b, norm_g, norm_b):
    B, S, D = x.shape
    xf = x.reshape(S, D)

    pos, w0, w1, tile_e = pl.pallas_call(
        _plan_kernel,
        grid=(4 * NUM_TILES,),
        in_specs=[
            _resident((S, D)),
            _resident((D, NUM_EXPERTS)),
            _resident((1, NUM_EXPERTS)),
        ],
        out_specs=[
            _resident((2 * S, 1)),
            _resident((S, 1)),
            _resident((S, 1)),
            _resident((1, 32)),
        ],
        out_shape=[jax.ShapeDtypeStruct((2 * S, 1), jnp.int32),
                   jax.ShapeDtypeStruct((S, 1), jnp.float32),
                   jax.ShapeDtypeStruct((S, 1), jnp.float32),
                   jax.ShapeDtypeStruct((1, 32), jnp.int32)],
        scratch_shapes=[pltpu.VMEM((S, NUM_EXPERTS), jnp.float32),
                        pltpu.VMEM((S, NUM_EXPERTS), jnp.float32),
                        pltpu.VMEM((1, NUM_EXPERTS), jnp.float32),
                        pltpu.VMEM((1, NUM_EXPERTS), jnp.float32)],
        compiler_params=pltpu.CompilerParams(
            dimension_semantics=("arbitrary",),
        ),
    )(xf, router_W, router_b.reshape(1, -1))

    pos_f = pos.reshape(2 * S)

    mesh = plsc.VectorSubcoreMesh(core_axis_name="c", subcore_axis_name="s",
                                  num_cores=2)
    xs = pl.kernel(
        _dispatch_kernel,
        mesh=mesh,
        out_type=jax.ShapeDtypeStruct((NP, D), jnp.float32),
        scratch_types=[pltpu.VMEM((2, CHUNK // 2), jnp.int32),
                       pltpu.VMEM((2, CHUNK // 2), jnp.int32),
                       pltpu.VMEM((CHUNK // 2, D), jnp.float32),
                       pltpu.SemaphoreType.DMA],
    )(pos_f, xf)

    ys = pl.pallas_call(
        _grouped_kernel,
        grid_spec=pltpu.PrefetchScalarGridSpec(
            num_scalar_prefetch=1,
            grid=(NT,),
            in_specs=[
                pl.BlockSpec((TG, D), lambda i, s: (i, 0)),
                pl.BlockSpec((1, DIM, EXPERT_DIM), lambda i, s: (s[i], 0, 0)),
                pl.BlockSpec((1, 1, EXPERT_DIM), lambda i, s: (s[i], 0, 0)),
                pl.BlockSpec((1, EXPERT_DIM, DIM), lambda i, s: (s[i], 0, 0)),
                pl.BlockSpec((1, 1, DIM), lambda i, s: (s[i], 0, 0)),
            ],
            out_specs=pl.BlockSpec((TG, D), lambda i, s: (i, 0)),
        ),
        out_shape=jax.ShapeDtypeStruct((NP, D), jnp.float32),
        compiler_params=pltpu.CompilerParams(
            dimension_semantics=("arbitrary",),
        ),
    )(tile_e.reshape(32), xs, expert_W1,
      expert_b1.reshape(NUM_EXPERTS, 1, EXPERT_DIM),
      expert_W2, expert_b2.reshape(NUM_EXPERTS, 1, DIM))

    z = pl.kernel(
        _unpermute_kernel,
        mesh=mesh,
        out_type=jax.ShapeDtypeStruct((2 * S, D), jnp.float32),
        scratch_types=[pltpu.VMEM((CHUNK,), jnp.int32),
                       pltpu.VMEM((CHUNK // 2, D), jnp.float32),
                       pltpu.SemaphoreType.DMA],
    )(pos_f, ys)

    out = pl.pallas_call(
        _micro_kernel,
        grid=(M_STEPS, NUM_TILES),
        in_specs=[
            _resident((2 * S, D)),
            _resident((S, 1)),
            _resident((S, 1)),
            _resident((D, NUM_MICROS)),
            _resident((1, NUM_MICROS)),
            _per_g((1, MG, DIM, MICRO_HID)),
            _per_g((1, MG, 1, MICRO_HID)),
            _per_g((1, MG, MICRO_HID, DIM)),
            _per_g((1, MG, DIM)),
            _per_g((1, MG, DIM)),
            _per_g((1, MG, DIM)),
            _resident((1, DIM)),
            _resident((1, DIM)),
        ],
        out_specs=_resident((S, D)),
        out_shape=jax.ShapeDtypeStruct((S, D), jnp.float32),
        scratch_shapes=[pltpu.VMEM((S, D), jnp.float32),
                        pltpu.VMEM((S, NUM_MICROS), jnp.float32)],
        compiler_params=pltpu.CompilerParams(
            dimension_semantics=("arbitrary", "arbitrary"),
        ),
    )(z, w0, w1, micro_router_W,
      micro_router_b.reshape(1, -1),
      micro_W1.reshape(M_STEPS, MG, DIM, MICRO_HID),
      micro_b1.reshape(M_STEPS, MG, 1, MICRO_HID),
      micro_W2.reshape(M_STEPS, MG, MICRO_HID, DIM),
      micro_b2.reshape(M_STEPS, MG, DIM),
      micro_ln_g.reshape(M_STEPS, MG, DIM),
      micro_ln_b.reshape(M_STEPS, MG, DIM),
      norm_g.reshape(1, -1), norm_b.reshape(1, -1))

    return out.reshape(B, S, D)
